# Initial kernel scaffold; baseline (speedup 1.0000x reference)
import jax
import jax.numpy as jnp
from jax.experimental import pallas as pl


def kernel(x, edge_index, enc1_Wl, enc1_Wr, enc1_b, enc2_Wl, enc2_Wr, enc2_b, dec1_Wl, dec1_Wr, dec1_b, dec2_Wl, dec2_Wr, dec2_b):
    raise NotImplementedError("write your pallas kernel here")


# speedup vs baseline: 5.1935x; 5.1935x over previous
import functools

import jax
import jax.numpy as jnp
from jax import lax
from jax.experimental import pallas as pl
from jax.experimental.pallas import tpu as pltpu
from jax.experimental.pallas import tpu_sc as plsc

N = 10000
E = 320000
IN_CH = 128
HID_CH = 128
LAT = 64
C = 128

NC = 2
NS = 16
NW = NC * NS
K = 128
CHUNKS = E // K
CPW = -(-CHUNKS // NW)
NP = 10240
RPT = NP // NS
ZROWS = 128
DEG_W = 128


def _agg_kernel():
    mesh = plsc.VectorSubcoreMesh(core_axis_name="c", subcore_axis_name="s")

    @functools.partial(
        pl.kernel,
        mesh=mesh,
        out_type=jax.ShapeDtypeStruct((NC * NP, C), jnp.float32),
        scratch_types=[
            pltpu.VMEM((ZROWS, C), jnp.float32),
            pltpu.VMEM((K,), jnp.int32),
            pltpu.VMEM((K,), jnp.int32),
            pltpu.VMEM((K, C), jnp.float32),
            pltpu.VMEM_SHARED((NP, C), jnp.float32),
            pltpu.SemaphoreType.DMA,
        ],
    )
    def k(h_hbm, src_hbm, dst_hbm, out_hbm, zbuf, src_v, dst_v, rows_v, acc_sh, sem):
        cid = lax.axis_index("c")
        sid = lax.axis_index("s")
        wid = sid * NC + cid

        def zrow(r, carry):
            for j in range(C // 16):
                zbuf[r, pl.ds(j * 16, 16)] = jnp.zeros((16,), jnp.float32)
            return carry
        lax.fori_loop(0, ZROWS, zrow, 0)
        for t in range(RPT // ZROWS):
            pltpu.sync_copy(zbuf, acc_sh.at[pl.ds(sid * RPT + t * ZROWS, ZROWS)])
        plsc.subcore_barrier()

        def body(i, carry):
            chunk = i * NW + wid

            @pl.when(chunk < CHUNKS)
            def _():
                base = chunk * K
                pltpu.sync_copy(src_hbm.at[pl.ds(base, K)], src_v)
                pltpu.async_copy(h_hbm.at[src_v], rows_v, sem).wait()
                pltpu.sync_copy(dst_hbm.at[pl.ds(base, K)], dst_v)
                pltpu.sync_copy(rows_v, acc_sh.at[dst_v], add=True)
            return carry
        lax.fori_loop(0, CPW, body, 0)
        plsc.subcore_barrier()

        pltpu.sync_copy(acc_sh.at[pl.ds(sid * RPT, RPT)],
                        out_hbm.at[pl.ds(cid * NP + sid * RPT, RPT)])

    return k


def _deg_kernel():
    mesh = plsc.VectorSubcoreMesh(core_axis_name="c", subcore_axis_name="s")

    @functools.partial(
        pl.kernel,
        mesh=mesh,
        out_type=jax.ShapeDtypeStruct((NC * NP, DEG_W), jnp.float32),
        scratch_types=[
            pltpu.VMEM((ZROWS, DEG_W), jnp.float32),
            pltpu.VMEM((K, DEG_W), jnp.float32),
            pltpu.VMEM((K,), jnp.int32),
            pltpu.VMEM_SHARED((NP, DEG_W), jnp.float32),
        ],
    )
    def k(dst_hbm, out_hbm, zbuf, ones_v, dst_v, acc_sh):
        cid = lax.axis_index("c")
        sid = lax.axis_index("s")
        wid = sid * NC + cid

        def zrow(r, carry):
            for j in range(DEG_W // 16):
                zbuf[r, pl.ds(j * 16, 16)] = jnp.zeros((16,), jnp.float32)
            return carry
        lax.fori_loop(0, ZROWS, zrow, 0)

        def orow(r, carry):
            for j in range(DEG_W // 16):
                ones_v[r, pl.ds(j * 16, 16)] = jnp.ones((16,), jnp.float32)
            return carry
        lax.fori_loop(0, K, orow, 0)
        for t in range(RPT // ZROWS):
            pltpu.sync_copy(zbuf, acc_sh.at[pl.ds(sid * RPT + t * ZROWS, ZROWS)])
        plsc.subcore_barrier()

        def body(i, carry):
            chunk = i * NW + wid

            @pl.when(chunk < CHUNKS)
            def _():
                pltpu.sync_copy(dst_hbm.at[pl.ds(chunk * K, K)], dst_v)
                pltpu.sync_copy(ones_v, acc_sh.at[dst_v], add=True)
            return carry
        lax.fori_loop(0, CPW, body, 0)
        plsc.subcore_barrier()

        pltpu.sync_copy(acc_sh.at[pl.ds(sid * RPT, RPT)],
                        out_hbm.at[pl.ds(cid * NP + sid * RPT, RPT)])

    return k


_R = 2000


def _rows_spec(width):
    return pl.BlockSpec((_R, width), lambda i: (i, 0))


def _full_spec(r, c):
    return pl.BlockSpec((r, c), lambda i: (0, 0))


def _tc_layer(relu):
    def body(a0, a1, d0r, d1r, hr, wl, wr, br, out):
        inv = 1.0 / jnp.maximum(d0r[:, :1] + d1r[:, :1], 1.0)
        mean = (a0[...] + a1[...]) * inv
        o = jnp.dot(mean, wl[...], preferred_element_type=jnp.float32)
        o += jnp.dot(hr[...], wr[...], preferred_element_type=jnp.float32)
        o = o + br[...]
        if relu:
            o = jnp.maximum(o, 0.0)
        out[...] = o

    return pl.pallas_call(
        body,
        grid=(N // _R,),
        in_specs=[_rows_spec(C), _rows_spec(C), _rows_spec(DEG_W),
                  _rows_spec(DEG_W), _rows_spec(C),
                  _full_spec(C, C), _full_spec(C, C), _full_spec(1, C)],
        out_specs=_rows_spec(C),
        out_shape=jax.ShapeDtypeStruct((N, C), jnp.float32),
    )


def _pad_w(w):
    return jnp.zeros((C, C), jnp.float32).at[:w.shape[0], :w.shape[1]].set(w)


def _pad_b(b):
    return jnp.zeros((1, C), jnp.float32).at[0, :b.shape[0]].set(b)


def kernel(x, edge_index, enc1_Wl, enc1_Wr, enc1_b, enc2_Wl, enc2_Wr, enc2_b,
           dec1_Wl, dec1_Wr, dec1_b, dec2_Wl, dec2_Wr, dec2_b):
    src = edge_index[0].astype(jnp.int32)
    dst = edge_index[1].astype(jnp.int32)

    deg = _deg_kernel()(dst)
    d0, d1 = deg[:N], deg[NP:NP + N]


    agg = _agg_kernel()
    relu_layer = _tc_layer(True)
    lin_layer = _tc_layer(False)

    weights = [
        (_pad_w(enc1_Wl), _pad_w(enc1_Wr), _pad_b(enc1_b)),
        (_pad_w(enc2_Wl), _pad_w(enc2_Wr), _pad_b(enc2_b)),
        (_pad_w(dec1_Wl), _pad_w(dec1_Wr), _pad_b(dec1_b)),
        (_pad_w(dec2_Wl), _pad_w(dec2_Wr), _pad_b(dec2_b)),
    ]

    h = x
    for i, (wl, wr, b) in enumerate(weights):
        a = agg(h, src, dst)
        layer = relu_layer if i in (0, 2) else lin_layer
        h = layer(a[:N], a[NP:NP + N], d0, d1, h, wl, wr, b)
    return h


# speedup vs baseline: 9.6227x; 1.8528x over previous
import functools

import jax
import jax.numpy as jnp
from jax import lax
from jax.experimental import pallas as pl
from jax.experimental.pallas import tpu as pltpu
from jax.experimental.pallas import tpu_sc as plsc

N = 10000
E = 320000
IN_CH = 128
HID_CH = 128
LAT = 64
C = 128

NC = 2
NS = 16
NW = NC * NS
K = 80
CPW = E // K // NW
CPWP = 128
NP = 10240
RPT = NP // NS
DEPTH = 4


def _agg_kernel():
    mesh = plsc.VectorSubcoreMesh(core_axis_name="c", subcore_axis_name="s")

    @functools.partial(
        pl.kernel,
        mesh=mesh,
        out_type=jax.ShapeDtypeStruct((NC * NP, C), jnp.float32),
        scratch_types=[
            pltpu.VMEM((CPWP, K), jnp.int32),
            pltpu.VMEM((K,), jnp.int32),
            pltpu.VMEM((K,), jnp.int32),
            pltpu.VMEM((K,), jnp.int32),
            pltpu.VMEM((K,), jnp.int32),
            pltpu.VMEM((K, C), jnp.float32),
            pltpu.VMEM((K, C), jnp.float32),
            pltpu.VMEM_SHARED((NP, C), jnp.float32),
            pltpu.SemaphoreType.DMA,
            pltpu.SemaphoreType.DMA,
            pltpu.SemaphoreType.DMA,
        ],
    )
    def k(h_hbm, pk_hbm, out_hbm, pidx, src_a, dst_a, src_b, dst_b,
          rows_a, rows_b, acc_sh, sem_a, sem_b, sem_i):
        cid = lax.axis_index("c")
        sid = lax.axis_index("s")
        wid = sid * NC + cid

        idx_cp = pltpu.async_copy(pk_hbm.at[pl.ds(wid * CPWP, CPWP)], pidx, sem_i)

        def zrow(r, carry):
            for j in range(C // 16):
                rows_a[r, pl.ds(j * 16, 16)] = jnp.zeros((16,), jnp.float32)
            return carry
        lax.fori_loop(0, K, zrow, 0)
        for t in range(RPT // K):
            pltpu.sync_copy(rows_a, acc_sh.at[pl.ds(sid * RPT + t * K, K)])
        idx_cp.wait()
        plsc.subcore_barrier()

        def unpack(i, sv, dv):
            @pl.when(i < CPW)
            def _():
                for j in range(K // 16):
                    v = pidx[i, pl.ds(j * 16, 16)]
                    sv[pl.ds(j * 16, 16)] = lax.bitwise_and(v, 16383)
                    dv[pl.ds(j * 16, 16)] = lax.shift_right_logical(v, 14)

        def gather(i, sv, buf, sem):
            @pl.when(i < CPW)
            def _():
                pltpu.async_copy(h_hbm.at[sv], buf, sem)

        def drain_scatter(i, sv, dv, buf, sem):
            @pl.when(i < CPW)
            def _():
                pltpu.make_async_copy(h_hbm.at[sv], buf, sem).wait()
                pltpu.sync_copy(buf, acc_sh.at[dv], add=True)

        unpack(0, src_a, dst_a)
        gather(0, src_a, rows_a, sem_a)

        def body(it, carry):
            g = it * 2
            unpack(g + 1, src_b, dst_b)
            gather(g + 1, src_b, rows_b, sem_b)
            drain_scatter(g, src_a, dst_a, rows_a, sem_a)
            unpack(g + 2, src_a, dst_a)
            gather(g + 2, src_a, rows_a, sem_a)
            drain_scatter(g + 1, src_b, dst_b, rows_b, sem_b)
            return carry
        lax.fori_loop(0, (CPW + 1) // 2, body, 0)
        plsc.subcore_barrier()

        pltpu.sync_copy(acc_sh.at[pl.ds(sid * RPT, RPT)],
                        out_hbm.at[pl.ds(cid * NP + sid * RPT, RPT)])

    return k


def _deg_kernel():
    mesh = plsc.VectorSubcoreMesh(core_axis_name="c", subcore_axis_name="s")

    @functools.partial(
        pl.kernel,
        mesh=mesh,
        out_type=jax.ShapeDtypeStruct((NC * NP, C), jnp.float32),
        scratch_types=[
            pltpu.VMEM((K, C), jnp.float32),
            pltpu.VMEM((CPWP, K), jnp.int32),
            pltpu.VMEM_SHARED((NP, C), jnp.float32),
            pltpu.SemaphoreType.DMA,
            pltpu.SemaphoreType.DMA,
        ],
    )
    def k(dst_hbm, out_hbm, ones_v, didx, acc_sh, sem_s, sem_i):
        cid = lax.axis_index("c")
        sid = lax.axis_index("s")
        wid = sid * NC + cid

        idx_cp = pltpu.async_copy(dst_hbm.at[pl.ds(wid * CPWP, CPWP)], didx, sem_i)

        def zrow(r, carry):
            for j in range(C // 16):
                ones_v[r, pl.ds(j * 16, 16)] = jnp.zeros((16,), jnp.float32)
            return carry
        lax.fori_loop(0, K, zrow, 0)
        for t in range(RPT // K):
            pltpu.sync_copy(ones_v, acc_sh.at[pl.ds(sid * RPT + t * K, K)])

        def orow(r, carry):
            for j in range(C // 16):
                ones_v[r, pl.ds(j * 16, 16)] = jnp.ones((16,), jnp.float32)
            return carry
        lax.fori_loop(0, K, orow, 0)
        idx_cp.wait()
        plsc.subcore_barrier()

        def fire(i):
            @pl.when(i < CPW)
            def _():
                pltpu.async_copy(ones_v, acc_sh.at[didx.at[i]], sem_s, add=True)

        def drain(i):
            @pl.when(i < CPW)
            def _():
                pltpu.make_async_copy(ones_v, acc_sh.at[didx.at[i]], sem_s).wait()

        for j in range(DEPTH):
            fire(j)

        def body(i, carry):
            drain(i)
            fire(i + DEPTH)
            return carry
        lax.fori_loop(0, CPW, body, 0)
        plsc.subcore_barrier()

        pltpu.sync_copy(acc_sh.at[pl.ds(sid * RPT, RPT)],
                        out_hbm.at[pl.ds(cid * NP + sid * RPT, RPT)])

    return k


_R = 2000


def _rows_spec(width):
    return pl.BlockSpec((_R, width), lambda i: (i, 0))


def _full_spec(r, c):
    return pl.BlockSpec((r, c), lambda i: (0, 0))


def _tc_layer(relu):
    def body(a0, a1, d0r, d1r, hr, wl, wr, br, out):
        inv = 1.0 / jnp.maximum(d0r[:, :1] + d1r[:, :1], 1.0)
        mean = (a0[...] + a1[...]) * inv
        o = jnp.dot(mean, wl[...], preferred_element_type=jnp.float32)
        o += jnp.dot(hr[...], wr[...], preferred_element_type=jnp.float32)
        o = o + br[...]
        if relu:
            o = jnp.maximum(o, 0.0)
        out[...] = o

    return pl.pallas_call(
        body,
        grid=(N // _R,),
        in_specs=[_rows_spec(C), _rows_spec(C), _rows_spec(C),
                  _rows_spec(C), _rows_spec(C),
                  _full_spec(C, C), _full_spec(C, C), _full_spec(1, C)],
        out_specs=_rows_spec(C),
        out_shape=jax.ShapeDtypeStruct((N, C), jnp.float32),
    )


def _pad_w(w):
    return jnp.zeros((C, C), jnp.float32).at[:w.shape[0], :w.shape[1]].set(w)


def _pad_b(b):
    return jnp.zeros((1, C), jnp.float32).at[0, :b.shape[0]].set(b)


def _worker_major(idx):
    p = idx.reshape(NW, CPW, K)
    p = jnp.concatenate([p, jnp.zeros((NW, CPWP - CPW, K), jnp.int32)], axis=1)
    return p.reshape(NW * CPWP, K)


def kernel(x, edge_index, enc1_Wl, enc1_Wr, enc1_b, enc2_Wl, enc2_Wr, enc2_b,
           dec1_Wl, dec1_Wr, dec1_b, dec2_Wl, dec2_Wr, dec2_b):
    src = edge_index[0].astype(jnp.int32)
    dst = edge_index[1].astype(jnp.int32)
    pk = _worker_major(jnp.left_shift(dst, 14) + src)
    dstw = _worker_major(dst)

    deg = _deg_kernel()(dstw)
    d0, d1 = deg[:N], deg[NP:NP + N]

    agg = _agg_kernel()
    relu_layer = _tc_layer(True)
    lin_layer = _tc_layer(False)

    weights = [
        (_pad_w(enc1_Wl), _pad_w(enc1_Wr), _pad_b(enc1_b)),
        (_pad_w(enc2_Wl), _pad_w(enc2_Wr), _pad_b(enc2_b)),
        (_pad_w(dec1_Wl), _pad_w(dec1_Wr), _pad_b(dec1_b)),
        (_pad_w(dec2_Wl), _pad_w(dec2_Wr), _pad_b(dec2_b)),
    ]

    h = x
    for i, (wl, wr, b) in enumerate(weights):
        a = agg(h, pk)
        layer = relu_layer if i in (0, 2) else lin_layer
        h = layer(a[:N], a[NP:NP + N], d0, d1, h, wl, wr, b)
    return h


# speedup vs baseline: 10.9849x; 1.1416x over previous
import functools

import jax
import jax.numpy as jnp
from jax import lax
from jax.experimental import pallas as pl
from jax.experimental.pallas import tpu as pltpu
from jax.experimental.pallas import tpu_sc as plsc

N = 10000
E = 320000
IN_CH = 128
HID_CH = 128
LAT = 64
C = 128

NC = 2
NS = 16
NW = NC * NS
K = 80
CPW = E // K // NW
CPWP = 128
NP = 10240
RPT = NP // NS
DEPTH = 4
SLOTS = 3


def _agg_kernel():
    mesh = plsc.VectorSubcoreMesh(core_axis_name="c", subcore_axis_name="s")

    @functools.partial(
        pl.kernel,
        mesh=mesh,
        out_type=jax.ShapeDtypeStruct((NC * NP, C), jnp.float32),
        scratch_types=[
            pltpu.VMEM((CPWP, K), jnp.int32),
            pltpu.VMEM((K, C), jnp.float32),
            pltpu.VMEM((K, C), jnp.float32),
            pltpu.VMEM((K, C), jnp.float32),
            pltpu.VMEM((K,), jnp.int32),
            pltpu.VMEM((K,), jnp.int32),
            pltpu.VMEM((K,), jnp.int32),
            pltpu.VMEM((K,), jnp.int32),
            pltpu.VMEM((K,), jnp.int32),
            pltpu.VMEM((K,), jnp.int32),
            pltpu.VMEM_SHARED((NP, C), jnp.float32),
            pltpu.SemaphoreType.DMA,
            pltpu.SemaphoreType.DMA,
            pltpu.SemaphoreType.DMA,
            pltpu.SemaphoreType.DMA,
            pltpu.SemaphoreType.DMA,
            pltpu.SemaphoreType.DMA,
            pltpu.SemaphoreType.DMA,
        ],
    )
    def k(h_hbm, pk_hbm, out_hbm, pidx, r0, r1, r2, sv0, dv0, sv1, dv1,
          sv2, dv2, acc_sh, g0, g1, g2, s0, s1, s2, sem_i):
        cid = lax.axis_index("c")
        sid = lax.axis_index("s")
        wid = sid * NC + cid
        rows = [r0, r1, r2]
        svs = [sv0, sv1, sv2]
        dvs = [dv0, dv1, dv2]
        gsem = [g0, g1, g2]
        ssem = [s0, s1, s2]

        idx_cp = pltpu.async_copy(pk_hbm.at[pl.ds(wid * CPWP, CPWP)], pidx, sem_i)

        def zrow(r, carry):
            for j in range(C // 16):
                r0[r, pl.ds(j * 16, 16)] = jnp.zeros((16,), jnp.float32)
            return carry
        lax.fori_loop(0, K, zrow, 0)
        for t in range(RPT // K):
            pltpu.sync_copy(r0, acc_sh.at[pl.ds(sid * RPT + t * K, K)])
        idx_cp.wait()
        plsc.subcore_barrier()

        def unpack(i, j):
            @pl.when(i < CPW)
            def _():
                for q in range(K // 16):
                    v = pidx[i, pl.ds(q * 16, 16)]
                    svs[j][pl.ds(q * 16, 16)] = lax.bitwise_and(v, 16383)
                    dvs[j][pl.ds(q * 16, 16)] = lax.shift_right_logical(v, 14)

        def issue_gather(i, j):
            @pl.when(i < CPW)
            def _():
                pltpu.async_copy(h_hbm.at[svs[j]], rows[j], gsem[j])

        def wait_gather(i, j):
            @pl.when(jnp.logical_and(i >= 0, i < CPW))
            def _():
                pltpu.make_async_copy(h_hbm.at[svs[j]], rows[j], gsem[j]).wait()

        def issue_scatter(i, j):
            @pl.when(jnp.logical_and(i >= 0, i < CPW))
            def _():
                pltpu.async_copy(rows[j], acc_sh.at[dvs[j]], ssem[j], add=True)

        def wait_scatter(i, j):
            @pl.when(jnp.logical_and(i >= 0, i < CPW))
            def _():
                pltpu.make_async_copy(rows[j], acc_sh.at[dvs[j]], ssem[j]).wait()

        def body(it, carry):
            base = it * SLOTS
            for j in range(SLOTS):
                ch = base + j
                wait_scatter(ch - SLOTS, j)
                unpack(ch, j)
                issue_gather(ch, j)
                jm1 = (j + SLOTS - 1) % SLOTS
                wait_gather(ch - 1, jm1)
                issue_scatter(ch - 1, jm1)
            return carry
        lax.fori_loop(0, (CPW + SLOTS) // SLOTS, body, 0)
        wait_scatter(CPW - 2, (CPW - 2) % SLOTS)
        wait_scatter(CPW - 1, (CPW - 1) % SLOTS)
        plsc.subcore_barrier()

        pltpu.sync_copy(acc_sh.at[pl.ds(sid * RPT, RPT)],
                        out_hbm.at[pl.ds(cid * NP + sid * RPT, RPT)])

    return k


def _deg_kernel():
    mesh = plsc.VectorSubcoreMesh(core_axis_name="c", subcore_axis_name="s")

    @functools.partial(
        pl.kernel,
        mesh=mesh,
        out_type=jax.ShapeDtypeStruct((NC * NP, C), jnp.float32),
        scratch_types=[
            pltpu.VMEM((K, C), jnp.float32),
            pltpu.VMEM((CPWP, K), jnp.int32),
            pltpu.VMEM_SHARED((NP, C), jnp.float32),
            pltpu.SemaphoreType.DMA,
            pltpu.SemaphoreType.DMA,
        ],
    )
    def k(dst_hbm, out_hbm, ones_v, didx, acc_sh, sem_s, sem_i):
        cid = lax.axis_index("c")
        sid = lax.axis_index("s")
        wid = sid * NC + cid

        idx_cp = pltpu.async_copy(dst_hbm.at[pl.ds(wid * CPWP, CPWP)], didx, sem_i)

        def zrow(r, carry):
            for j in range(C // 16):
                ones_v[r, pl.ds(j * 16, 16)] = jnp.zeros((16,), jnp.float32)
            return carry
        lax.fori_loop(0, K, zrow, 0)
        for t in range(RPT // K):
            pltpu.sync_copy(ones_v, acc_sh.at[pl.ds(sid * RPT + t * K, K)])

        def orow(r, carry):
            for j in range(C // 16):
                ones_v[r, pl.ds(j * 16, 16)] = jnp.ones((16,), jnp.float32)
            return carry
        lax.fori_loop(0, K, orow, 0)
        idx_cp.wait()
        plsc.subcore_barrier()

        def fire(i):
            @pl.when(i < CPW)
            def _():
                pltpu.async_copy(ones_v, acc_sh.at[didx.at[i]], sem_s, add=True)

        def drain(i):
            @pl.when(i < CPW)
            def _():
                pltpu.make_async_copy(ones_v, acc_sh.at[didx.at[i]], sem_s).wait()

        for j in range(DEPTH):
            fire(j)

        def body(i, carry):
            drain(i)
            fire(i + DEPTH)
            return carry
        lax.fori_loop(0, CPW, body, 0)
        plsc.subcore_barrier()

        pltpu.sync_copy(acc_sh.at[pl.ds(sid * RPT, RPT)],
                        out_hbm.at[pl.ds(cid * NP + sid * RPT, RPT)])

    return k


_R = 2000


def _rows_spec(width):
    return pl.BlockSpec((_R, width), lambda i: (i, 0))


def _full_spec(r, c):
    return pl.BlockSpec((r, c), lambda i: (0, 0))


def _tc_layer(relu):
    def body(a0, a1, d0r, d1r, hr, wl, wr, br, out):
        inv = 1.0 / jnp.maximum(d0r[...] + d1r[...], 1.0)
        mean = (a0[...] + a1[...]) * inv
        o = jnp.dot(mean, wl[...], preferred_element_type=jnp.float32)
        o += jnp.dot(hr[...], wr[...], preferred_element_type=jnp.float32)
        o = o + br[...]
        if relu:
            o = jnp.maximum(o, 0.0)
        out[...] = o

    return pl.pallas_call(
        body,
        grid=(N // _R,),
        in_specs=[_rows_spec(C), _rows_spec(C), _rows_spec(1),
                  _rows_spec(1), _rows_spec(C),
                  _full_spec(C, C), _full_spec(C, C), _full_spec(1, C)],
        out_specs=_rows_spec(C),
        out_shape=jax.ShapeDtypeStruct((N, C), jnp.float32),
    )


def _pad_w(w):
    return jnp.zeros((C, C), jnp.float32).at[:w.shape[0], :w.shape[1]].set(w)


def _pad_b(b):
    return jnp.zeros((1, C), jnp.float32).at[0, :b.shape[0]].set(b)


def _worker_major(idx):
    p = idx.reshape(NW, CPW, K)
    p = jnp.concatenate([p, jnp.zeros((NW, CPWP - CPW, K), jnp.int32)], axis=1)
    return p.reshape(NW * CPWP, K)


def kernel(x, edge_index, enc1_Wl, enc1_Wr, enc1_b, enc2_Wl, enc2_Wr, enc2_b,
           dec1_Wl, dec1_Wr, dec1_b, dec2_Wl, dec2_Wr, dec2_b):
    src = edge_index[0].astype(jnp.int32)
    dst = edge_index[1].astype(jnp.int32)
    pk = _worker_major(jnp.left_shift(dst, 14) + src)
    dstw = _worker_major(dst)

    deg = _deg_kernel()(dstw)
    d0 = deg[:N, :1]
    d1 = deg[NP:NP + N, :1]

    agg = _agg_kernel()
    relu_layer = _tc_layer(True)
    lin_layer = _tc_layer(False)

    weights = [
        (_pad_w(enc1_Wl), _pad_w(enc1_Wr), _pad_b(enc1_b)),
        (_pad_w(enc2_Wl), _pad_w(enc2_Wr), _pad_b(enc2_b)),
        (_pad_w(dec1_Wl), _pad_w(dec1_Wr), _pad_b(dec1_b)),
        (_pad_w(dec2_Wl), _pad_w(dec2_Wr), _pad_b(dec2_b)),
    ]

    h = x
    for i, (wl, wr, b) in enumerate(weights):
        a = agg(h, pk)
        layer = relu_layer if i in (0, 2) else lin_layer
        h = layer(a[:N], a[NP:NP + N], d0, d1, h, wl, wr, b)
    return h


# speedup vs baseline: 11.8405x; 1.0779x over previous
import functools

import jax
import jax.numpy as jnp
from jax import lax
from jax.experimental import pallas as pl
from jax.experimental.pallas import tpu as pltpu
from jax.experimental.pallas import tpu_sc as plsc

N = 10000
E = 320000
IN_CH = 128
HID_CH = 128
LAT = 64
C = 128

NC = 2
NS = 16
NW = NC * NS
K = 80
CPW = E // K // NW
CPWP = 128
NP = 10240
RPT = NP // NS
DEPTH = 4
SLOTS = 3


def _agg_kernel():
    mesh = plsc.VectorSubcoreMesh(core_axis_name="c", subcore_axis_name="s")

    @functools.partial(
        pl.kernel,
        mesh=mesh,
        out_type=jax.ShapeDtypeStruct((NC * NP, C), jnp.float32),
        scratch_types=[
            pltpu.VMEM((CPWP, K), jnp.int32),
            pltpu.VMEM((K, C), jnp.float32),
            pltpu.VMEM((K, C), jnp.float32),
            pltpu.VMEM((K, C), jnp.float32),
            pltpu.VMEM((K,), jnp.int32),
            pltpu.VMEM((K,), jnp.int32),
            pltpu.VMEM((K,), jnp.int32),
            pltpu.VMEM((K,), jnp.int32),
            pltpu.VMEM((K,), jnp.int32),
            pltpu.VMEM((K,), jnp.int32),
            pltpu.VMEM_SHARED((NP, C), jnp.float32),
            pltpu.SemaphoreType.DMA,
            pltpu.SemaphoreType.DMA,
            pltpu.SemaphoreType.DMA,
            pltpu.SemaphoreType.DMA,
            pltpu.SemaphoreType.DMA,
            pltpu.SemaphoreType.DMA,
            pltpu.SemaphoreType.DMA,
        ],
    )
    def k(h_hbm, pk_hbm, out_hbm, pidx, r0, r1, r2, sv0, dv0, sv1, dv1,
          sv2, dv2, acc_sh, g0, g1, g2, s0, s1, s2, sem_i):
        cid = lax.axis_index("c")
        sid = lax.axis_index("s")
        wid = sid * NC + cid
        rows = [r0, r1, r2]
        svs = [sv0, sv1, sv2]
        dvs = [dv0, dv1, dv2]
        gsem = [g0, g1, g2]
        ssem = [s0, s1, s2]

        idx_cp = pltpu.async_copy(pk_hbm.at[pl.ds(wid * CPWP, CPWP)], pidx, sem_i)

        def zrow(r, carry):
            for j in range(C // 16):
                r0[r, pl.ds(j * 16, 16)] = jnp.zeros((16,), jnp.float32)
            return carry
        lax.fori_loop(0, K, zrow, 0)
        zcps = [pltpu.async_copy(r0, acc_sh.at[pl.ds(sid * RPT + t * K, K)], g1)
                for t in range(RPT // K)]
        for cp in zcps:
            cp.wait()
        idx_cp.wait()
        plsc.subcore_barrier()

        def unpack(i, j):
            @pl.when(i < CPW)
            def _():
                for q in range(K // 16):
                    v = pidx[i, pl.ds(q * 16, 16)]
                    svs[j][pl.ds(q * 16, 16)] = lax.bitwise_and(v, 16383)
                    dvs[j][pl.ds(q * 16, 16)] = lax.shift_right_logical(v, 14)

        def issue_gather(i, j):
            @pl.when(i < CPW)
            def _():
                pltpu.async_copy(h_hbm.at[svs[j]], rows[j], gsem[j])

        def wait_gather(i, j):
            @pl.when(jnp.logical_and(i >= 0, i < CPW))
            def _():
                pltpu.make_async_copy(h_hbm.at[svs[j]], rows[j], gsem[j]).wait()

        def issue_scatter(i, j):
            @pl.when(jnp.logical_and(i >= 0, i < CPW))
            def _():
                pltpu.async_copy(rows[j], acc_sh.at[dvs[j]], ssem[j], add=True)

        def wait_scatter(i, j):
            @pl.when(jnp.logical_and(i >= 0, i < CPW))
            def _():
                pltpu.make_async_copy(rows[j], acc_sh.at[dvs[j]], ssem[j]).wait()

        def body(it, carry):
            base = it * SLOTS
            for j in range(SLOTS):
                ch = base + j
                wait_scatter(ch - SLOTS, j)
                unpack(ch, j)
                issue_gather(ch, j)
                jm1 = (j + SLOTS - 1) % SLOTS
                wait_gather(ch - 1, jm1)
                issue_scatter(ch - 1, jm1)
            return carry
        lax.fori_loop(0, (CPW + SLOTS) // SLOTS, body, 0)
        wait_scatter(CPW - 2, (CPW - 2) % SLOTS)
        wait_scatter(CPW - 1, (CPW - 1) % SLOTS)
        plsc.subcore_barrier()

        pltpu.sync_copy(acc_sh.at[pl.ds(sid * RPT, RPT)],
                        out_hbm.at[pl.ds(cid * NP + sid * RPT, RPT)])

    return k


DEG_W = 32


def _deg_kernel():
    mesh = plsc.VectorSubcoreMesh(core_axis_name="c", subcore_axis_name="s")

    @functools.partial(
        pl.kernel,
        mesh=mesh,
        out_type=jax.ShapeDtypeStruct((NC * NP, DEG_W), jnp.float32),
        scratch_types=[
            pltpu.VMEM((K, DEG_W), jnp.float32),
            pltpu.VMEM((CPWP, K), jnp.int32),
            pltpu.VMEM_SHARED((NP, DEG_W), jnp.float32),
            pltpu.SemaphoreType.DMA,
            pltpu.SemaphoreType.DMA,
        ],
    )
    def k(dst_hbm, out_hbm, ones_v, didx, acc_sh, sem_s, sem_i):
        cid = lax.axis_index("c")
        sid = lax.axis_index("s")
        wid = sid * NC + cid

        idx_cp = pltpu.async_copy(dst_hbm.at[pl.ds(wid * CPWP, CPWP)], didx, sem_i)

        def zrow(r, carry):
            for j in range(DEG_W // 16):
                ones_v[r, pl.ds(j * 16, 16)] = jnp.zeros((16,), jnp.float32)
            return carry
        lax.fori_loop(0, K, zrow, 0)
        zcps = [pltpu.async_copy(ones_v, acc_sh.at[pl.ds(sid * RPT + t * K, K)],
                                 sem_s) for t in range(RPT // K)]
        for cp in zcps:
            cp.wait()

        def orow(r, carry):
            for j in range(DEG_W // 16):
                ones_v[r, pl.ds(j * 16, 16)] = jnp.ones((16,), jnp.float32)
            return carry
        lax.fori_loop(0, K, orow, 0)
        idx_cp.wait()
        plsc.subcore_barrier()

        def fire(i):
            @pl.when(i < CPW)
            def _():
                pltpu.async_copy(ones_v, acc_sh.at[didx.at[i]], sem_s, add=True)

        def drain(i):
            @pl.when(i < CPW)
            def _():
                pltpu.make_async_copy(ones_v, acc_sh.at[didx.at[i]], sem_s).wait()

        for j in range(DEPTH):
            fire(j)

        def body(i, carry):
            drain(i)
            fire(i + DEPTH)
            return carry
        lax.fori_loop(0, CPW, body, 0)
        plsc.subcore_barrier()

        pltpu.sync_copy(acc_sh.at[pl.ds(sid * RPT, RPT)],
                        out_hbm.at[pl.ds(cid * NP + sid * RPT, RPT)])

    return k


_R = 2000


def _rows_spec(width):
    return pl.BlockSpec((_R, width), lambda i: (i, 0))


def _full_spec(r, c):
    return pl.BlockSpec((r, c), lambda i: (0, 0))


def _tc_layer(relu):
    def body(a0, a1, d0r, d1r, hr, wl, wr, br, out):
        inv = 1.0 / jnp.maximum(d0r[...] + d1r[...], 1.0)
        mean = (a0[...] + a1[...]) * inv
        o = jnp.dot(mean, wl[...], preferred_element_type=jnp.float32)
        o += jnp.dot(hr[...], wr[...], preferred_element_type=jnp.float32)
        o = o + br[...]
        if relu:
            o = jnp.maximum(o, 0.0)
        out[...] = o

    return pl.pallas_call(
        body,
        grid=(N // _R,),
        in_specs=[_rows_spec(C), _rows_spec(C), _rows_spec(1),
                  _rows_spec(1), _rows_spec(C),
                  _full_spec(C, C), _full_spec(C, C), _full_spec(1, C)],
        out_specs=_rows_spec(C),
        out_shape=jax.ShapeDtypeStruct((N, C), jnp.float32),
    )


def _pad_w(w):
    return jnp.zeros((C, C), jnp.float32).at[:w.shape[0], :w.shape[1]].set(w)


def _pad_b(b):
    return jnp.zeros((1, C), jnp.float32).at[0, :b.shape[0]].set(b)


def _worker_major(idx):
    p = idx.reshape(NW, CPW, K)
    p = jnp.concatenate([p, jnp.zeros((NW, CPWP - CPW, K), jnp.int32)], axis=1)
    return p.reshape(NW * CPWP, K)


def kernel(x, edge_index, enc1_Wl, enc1_Wr, enc1_b, enc2_Wl, enc2_Wr, enc2_b,
           dec1_Wl, dec1_Wr, dec1_b, dec2_Wl, dec2_Wr, dec2_b):
    src = edge_index[0].astype(jnp.int32)
    dst = edge_index[1].astype(jnp.int32)
    pk = _worker_major(jnp.left_shift(dst, 14) + src)
    dstw = _worker_major(dst)

    deg = _deg_kernel()(dstw)
    d0 = deg[:N, :1]
    d1 = deg[NP:NP + N, :1]

    agg = _agg_kernel()
    relu_layer = _tc_layer(True)
    lin_layer = _tc_layer(False)

    weights = [
        (_pad_w(enc1_Wl), _pad_w(enc1_Wr), _pad_b(enc1_b)),
        (_pad_w(enc2_Wl), _pad_w(enc2_Wr), _pad_b(enc2_b)),
        (_pad_w(dec1_Wl), _pad_w(dec1_Wr), _pad_b(dec1_b)),
        (_pad_w(dec2_Wl), _pad_w(dec2_Wr), _pad_b(dec2_b)),
    ]

    h = x
    for i, (wl, wr, b) in enumerate(weights):
        a = agg(h, pk)
        layer = relu_layer if i in (0, 2) else lin_layer
        h = layer(a[:N], a[NP:NP + N], d0, d1, h, wl, wr, b)
    return h
